# Initial kernel scaffold; baseline (speedup 1.0000x reference)
import jax
import jax.numpy as jnp
from jax.experimental import pallas as pl


def kernel(x, edge_index, node_types, edge_types, W_in, b_in, W_rel, W_hid, b_hid, W_out, b_out, W_log, b_log):
    raise NotImplementedError("write your pallas kernel here")


# speedup vs baseline: 12.0686x; 12.0686x over previous
import functools

import jax
import jax.numpy as jnp
from jax import lax
from jax.experimental import pallas as pl
from jax.experimental.pallas import tpu as pltpu
from jax.experimental.pallas import tpu_sc as plsc

_CH = 128
_NC = 2
_NS = 16
_NTILE = _NC * _NS


def _leaky(v):
    return jnp.where(v >= 0, v, 0.01 * v)


def _input_transform(x, wcat, types2d, b_in, NT, H):
    N, D = x.shape

    def body(x_ref, w_ref, t_ref, b_ref, o_ref):
        y = jnp.dot(x_ref[...], w_ref[...], preferred_element_type=jnp.float32)
        t = t_ref[...]
        acc = jnp.zeros((N, H), jnp.float32)
        for tt in range(NT):
            acc = acc + jnp.where(t == tt, y[:, tt * H:(tt + 1) * H], 0.0)
        o_ref[...] = _leaky(acc + b_ref[...])

    return pl.pallas_call(
        body,
        out_shape=jax.ShapeDtypeStruct((N, H), jnp.float32),
    )(x, wcat, types2d, b_in.reshape(1, H))


def _make_mp_round(N, H, KPT, NPAD):
    mesh = plsc.VectorSubcoreMesh(core_axis_name="c", subcore_axis_name="s")
    rpt = NPAD // _NS

    @functools.partial(
        pl.kernel,
        out_type=jax.ShapeDtypeStruct((_NC * NPAD, H), jnp.float32),
        mesh=mesh,
        scratch_types=[
            pltpu.VMEM((KPT, _CH), jnp.int32),
            pltpu.VMEM((KPT, _CH), jnp.int32),
            pltpu.VMEM((_CH, H), jnp.float32),
            pltpu.VMEM((_CH, H), jnp.float32),
            pltpu.VMEM_SHARED((NPAD, H), jnp.float32),
            pltpu.SemaphoreType.DMA,
            pltpu.SemaphoreType.DMA,
        ],
        compiler_params=pltpu.CompilerParams(use_tc_tiling_on_sc=False),
    )
    def mp(h_hbm, src_hbm, dst_hbm, zero_hbm, out_hbm,
           src_v, dst_v, rows0, rows1, agg, sem0, sem1):
        c = lax.axis_index("c")
        s = lax.axis_index("s")
        wid = c * _NS + s
        pltpu.sync_copy(zero_hbm.at[pl.ds(s * rpt, rpt)],
                        agg.at[pl.ds(s * rpt, rpt)])
        pltpu.sync_copy(src_hbm.at[pl.ds(wid * KPT, KPT)], src_v)
        pltpu.sync_copy(dst_hbm.at[pl.ds(wid * KPT, KPT)], dst_v)
        plsc.subcore_barrier()

        pltpu.async_copy(h_hbm.at[src_v.at[0]], rows0, sem0)

        def pair(i, carry):
            j = 2 * i
            pltpu.make_async_copy(h_hbm.at[src_v.at[j]], rows0, sem0).wait()
            pltpu.async_copy(h_hbm.at[src_v.at[j + 1]], rows1, sem1)
            pltpu.sync_copy(rows0, agg.at[dst_v.at[j]], add=True)
            pltpu.make_async_copy(h_hbm.at[src_v.at[j + 1]], rows1, sem1).wait()

            @pl.when(j + 2 < KPT)
            def _():
                pltpu.async_copy(h_hbm.at[src_v.at[j + 2]], rows0, sem0)

            pltpu.sync_copy(rows1, agg.at[dst_v.at[j + 1]], add=True)
            return carry

        lax.fori_loop(0, KPT // 2, pair, 0)
        plsc.subcore_barrier()
        pltpu.sync_copy(agg.at[pl.ds(s * rpt, rpt)],
                        out_hbm.at[pl.ds(c * NPAD + s * rpt, rpt)])

    return mp


def _mid(p, wrel, N, NPAD, H):
    def body(p_ref, w_ref, o_ref):
        v = p_ref[:N, :] + p_ref[NPAD:NPAD + N, :]
        o_ref[...] = _leaky(
            jnp.dot(v, w_ref[...], preferred_element_type=jnp.float32))

    return pl.pallas_call(
        body,
        out_shape=jax.ShapeDtypeStruct((N, H), jnp.float32),
    )(p, wrel)


def _head(q, whid, bhid, wout, bout, wlog_t, blog, N, NPAD, H, OUT):
    def body(q_ref, wh_ref, bh_ref, wo_ref, bo_ref, wl_ref, bl_ref,
             out_ref, emb_ref):
        v = q_ref[:N, :] + q_ref[NPAD:NPAD + N, :]
        h2 = _leaky(
            jnp.dot(v, wh_ref[...], preferred_element_type=jnp.float32)
            + bh_ref[...])
        hn = jnp.dot(h2, wo_ref[...], preferred_element_type=jnp.float32) \
            + bo_ref[...]
        g = jnp.mean(hn, axis=0, keepdims=True)
        emb = _leaky(g)
        logit = jnp.sum(emb * wl_ref[...], axis=1, keepdims=True) + bl_ref[...]
        out_ref[...] = jax.nn.sigmoid(logit)
        emb_ref[...] = emb

    return pl.pallas_call(
        body,
        out_shape=(jax.ShapeDtypeStruct((1, 1), jnp.float32),
                   jax.ShapeDtypeStruct((1, OUT), jnp.float32)),
    )(q, whid, bhid.reshape(1, H), wout, bout.reshape(1, OUT),
      wlog_t, blog.reshape(1, 1))


def kernel(x, edge_index, node_types, edge_types, W_in, b_in, W_rel,
           W_hid, b_hid, W_out, b_out, W_log, b_log):
    N, D = x.shape
    NT, _, H = W_in.shape
    OUT = W_out.shape[1]
    E = edge_index.shape[1]

    wcat = jnp.transpose(W_in, (1, 0, 2)).reshape(D, NT * H)
    h = _input_transform(x, wcat, node_types.reshape(N, 1), b_in, NT, H)

    nchunk = -(-E // _CH)
    kpt = -(-nchunk // _NTILE)
    kpt = kpt + (kpt % 2)
    epad = kpt * _NTILE * _CH
    src = jnp.concatenate(
        [edge_index[0], jnp.zeros((epad - E,), jnp.int32)]).reshape(-1, _CH)
    dst = jnp.concatenate(
        [edge_index[1], jnp.full((epad - E,), N, jnp.int32)]).reshape(-1, _CH)
    npad = -(-N // (_NS * 8)) * (_NS * 8)
    zeros = jnp.zeros((npad, H), jnp.float32)

    mp = _make_mp_round(N, H, kpt, npad)
    p = mp(h, src, dst, zeros)
    h1 = _mid(p, W_rel[0], N, npad, H)
    q = mp(h1, src, dst, zeros)
    out, emb = _head(q, W_hid, b_hid, W_out, b_out,
                     jnp.transpose(W_log), b_log, N, npad, H, OUT)
    return out, emb.reshape(OUT)


# speedup vs baseline: 15.4852x; 1.2831x over previous
import functools

import jax
import jax.numpy as jnp
from jax import lax
from jax.experimental import pallas as pl
from jax.experimental.pallas import tpu as pltpu
from jax.experimental.pallas import tpu_sc as plsc

_CH = 128
_NC = 2
_NS = 16
_NTILE = _NC * _NS


def _leaky(v):
    return jnp.where(v >= 0, v, 0.01 * v)


def _input_transform(x, wcat, types2d, b_in, NT, H):
    N, D = x.shape

    def body(x_ref, w_ref, t_ref, b_ref, o_ref):
        y = jnp.dot(x_ref[...], w_ref[...], preferred_element_type=jnp.float32)
        t = t_ref[...]
        acc = jnp.zeros((N, H), jnp.float32)
        for tt in range(NT):
            acc = acc + jnp.where(t == tt, y[:, tt * H:(tt + 1) * H], 0.0)
        o_ref[...] = _leaky(acc + b_ref[...])

    return pl.pallas_call(
        body,
        out_shape=jax.ShapeDtypeStruct((N, H), jnp.float32),
    )(x, wcat, types2d, b_in.reshape(1, H))


def _make_mp_round(N, H, KPT, NPAD):
    mesh = plsc.VectorSubcoreMesh(core_axis_name="c", subcore_axis_name="s")
    rpt = NPAD // _NS

    NB = 8
    AHEAD = 4
    assert KPT % NB == 0

    @functools.partial(
        pl.kernel,
        out_type=jax.ShapeDtypeStruct((_NC * NPAD, H), jnp.float32),
        mesh=mesh,
        scratch_types=(
            [pltpu.VMEM((KPT, _CH), jnp.int32)] * 2
            + [pltpu.VMEM((_CH, H), jnp.float32)] * NB
            + [pltpu.VMEM_SHARED((NPAD, H), jnp.float32)]
            + [pltpu.SemaphoreType.DMA] * (2 * NB)
        ),
        compiler_params=pltpu.CompilerParams(use_tc_tiling_on_sc=False),
    )
    def mp(h_hbm, src_hbm, dst_hbm, zero_hbm, out_hbm, src_v, dst_v, *rest):
        rows = rest[:NB]
        agg = rest[NB]
        gsem = rest[NB + 1:2 * NB + 1]
        ssem = rest[2 * NB + 1:]
        c = lax.axis_index("c")
        s = lax.axis_index("s")
        wid = c * _NS + s
        pltpu.sync_copy(zero_hbm.at[pl.ds(s * rpt, rpt)],
                        agg.at[pl.ds(s * rpt, rpt)])
        pltpu.sync_copy(src_hbm.at[pl.ds(wid * KPT, KPT)], src_v)
        pltpu.sync_copy(dst_hbm.at[pl.ds(wid * KPT, KPT)], dst_v)
        plsc.subcore_barrier()

        for u in range(AHEAD):
            pltpu.async_copy(h_hbm.at[src_v.at[u]], rows[u], gsem[u])

        def outer(o, carry):
            base = NB * o
            for u in range(NB):
                i = base + u
                v = (u + AHEAD) % NB
                pltpu.make_async_copy(
                    h_hbm.at[src_v.at[i]], rows[u], gsem[u]).wait()
                pltpu.async_copy(
                    rows[u], agg.at[dst_v.at[i]], ssem[u], add=True)

                @pl.when(i - AHEAD >= 0)
                def _():
                    pltpu.make_async_copy(
                        rows[v], agg.at[dst_v.at[i - AHEAD]], ssem[v]).wait()

                @pl.when(i + AHEAD < KPT)
                def _():
                    pltpu.async_copy(
                        h_hbm.at[src_v.at[i + AHEAD]], rows[v], gsem[v])
            return carry

        lax.fori_loop(0, KPT // NB, outer, 0)
        for k in range(AHEAD):
            i = KPT - AHEAD + k
            u = i % NB
            pltpu.make_async_copy(
                rows[u], agg.at[dst_v.at[i]], ssem[u]).wait()
        plsc.subcore_barrier()
        pltpu.sync_copy(agg.at[pl.ds(s * rpt, rpt)],
                        out_hbm.at[pl.ds(c * NPAD + s * rpt, rpt)])

    return mp


def _mid(p, wrel, N, NPAD, H):
    def body(p_ref, w_ref, o_ref):
        v = p_ref[:N, :] + p_ref[NPAD:NPAD + N, :]
        o_ref[...] = _leaky(
            jnp.dot(v, w_ref[...], preferred_element_type=jnp.float32))

    return pl.pallas_call(
        body,
        out_shape=jax.ShapeDtypeStruct((N, H), jnp.float32),
    )(p, wrel)


def _head(q, whid, bhid, wout, bout, wlog_t, blog, N, NPAD, H, OUT):
    def body(q_ref, wh_ref, bh_ref, wo_ref, bo_ref, wl_ref, bl_ref,
             out_ref, emb_ref):
        v = q_ref[:N, :] + q_ref[NPAD:NPAD + N, :]
        h2 = _leaky(
            jnp.dot(v, wh_ref[...], preferred_element_type=jnp.float32)
            + bh_ref[...])
        hn = jnp.dot(h2, wo_ref[...], preferred_element_type=jnp.float32) \
            + bo_ref[...]
        g = jnp.mean(hn, axis=0, keepdims=True)
        emb = _leaky(g)
        logit = jnp.sum(emb * wl_ref[...], axis=1, keepdims=True) + bl_ref[...]
        out_ref[...] = jax.nn.sigmoid(logit)
        emb_ref[...] = emb

    return pl.pallas_call(
        body,
        out_shape=(jax.ShapeDtypeStruct((1, 1), jnp.float32),
                   jax.ShapeDtypeStruct((1, OUT), jnp.float32)),
    )(q, whid, bhid.reshape(1, H), wout, bout.reshape(1, OUT),
      wlog_t, blog.reshape(1, 1))


def kernel(x, edge_index, node_types, edge_types, W_in, b_in, W_rel,
           W_hid, b_hid, W_out, b_out, W_log, b_log):
    N, D = x.shape
    NT, _, H = W_in.shape
    OUT = W_out.shape[1]
    E = edge_index.shape[1]

    wcat = jnp.transpose(W_in, (1, 0, 2)).reshape(D, NT * H)
    h = _input_transform(x, wcat, node_types.reshape(N, 1), b_in, NT, H)

    nchunk = -(-E // _CH)
    kpt = -(-nchunk // _NTILE)
    kpt = -(-kpt // 8) * 8
    epad = kpt * _NTILE * _CH
    src = jnp.concatenate(
        [edge_index[0], jnp.zeros((epad - E,), jnp.int32)]).reshape(-1, _CH)
    dst = jnp.concatenate(
        [edge_index[1], jnp.full((epad - E,), N, jnp.int32)]).reshape(-1, _CH)
    npad = -(-N // (_NS * 8)) * (_NS * 8)
    zeros = jnp.zeros((npad, H), jnp.float32)

    mp = _make_mp_round(N, H, kpt, npad)
    p = mp(h, src, dst, zeros)
    h1 = _mid(p, W_rel[0], N, npad, H)
    q = mp(h1, src, dst, zeros)
    out, emb = _head(q, W_hid, b_hid, W_out, b_out,
                     jnp.transpose(W_log), b_log, N, npad, H, OUT)
    return out, emb.reshape(OUT)
